# Initial kernel scaffold; baseline (speedup 1.0000x reference)
import functools

import jax
import jax.numpy as jnp
from jax import lax
from jax.experimental import pallas as pl
from jax.experimental.pallas import tpu as pltpu
from jax.experimental.pallas import tpu_sc as plsc

_NC = 2
_NS = 16
_NW = _NC * _NS
_L = 16
_BIG = 1 << 20


@functools.lru_cache(maxsize=None)
def _build(T, V, D, N, S):
    CHUNK = T // _NW
    HALF = 128
    NSUB = CHUNK // HALF
    NG = CHUNK // _L
    NCOL = D // _L
    assert T % _NW == 0 and CHUNK % HALF == 0 and D % _L == 0
    assert S % CHUNK == 0
    assert N <= _L

    mesh = plsc.VectorSubcoreMesh(core_axis_name="c", subcore_axis_name="s")

    @functools.partial(
        pl.kernel,
        mesh=mesh,
        out_type=jax.ShapeDtypeStruct((T, D), jnp.float32),
        scratch_types=[
            pltpu.VMEM((CHUNK,), jnp.int32),
            pltpu.VMEM((_L,), jnp.int32),
            pltpu.VMEM((N, D), jnp.float32),
            pltpu.VMEM((CHUNK, D), jnp.float32),
            pltpu.SemaphoreType.DMA,
        ],
    )
    def _soft_embed(tok_hbm, wte_hbm, learned_hbm, out_hbm,
                    tok_v, look_v, learned_v, rows_v, sem):
        wid = lax.axis_index("s") * _NC + lax.axis_index("c")
        base = wid * CHUNK

        pltpu.sync_copy(tok_hbm.at[pl.ds(base, CHUNK)], tok_v)

        @pl.when(wid < _NW - 1)
        def _():
            pltpu.sync_copy(tok_hbm.at[pl.ds(base + CHUNK, _L)], look_v)

        @pl.when(wid == _NW - 1)
        def _():
            look_v[...] = jnp.ones((_L,), jnp.int32)

        cps = [
            pltpu.async_copy(
                wte_hbm.at[tok_v.at[pl.ds(h * HALF, HALF)]],
                rows_v.at[pl.ds(h * HALF, HALF)],
                sem,
            )
            for h in range(NSUB)
        ]

        pltpu.sync_copy(learned_hbm, learned_v)

        acc = jnp.full((_L,), _BIG, jnp.int32)
        lane = lax.iota(jnp.int32, _L)
        for q in range(NG):
            t = tok_v[pl.ds(q * _L, _L)]
            acc = jnp.minimum(acc, jnp.where(t == 0, lane + q * _L, _BIG))
        t = look_v[...]
        acc = jnp.minimum(acc, jnp.where(t == 0, lane + CHUNK, _BIG))
        loc = jnp.min(acc)

        for cp in cps:
            cp.wait()

        for o in range(N):
            p = loc - N + o
            valid = jnp.logical_and(p >= 0, p < CHUNK)
            pv = jnp.full((_L,), p, jnp.int32)
            mk = jnp.full((_L,), valid)
            for c in range(NCOL):
                cols = lane + c * _L
                src = learned_v[o, pl.ds(c * _L, _L)]
                plsc.store_scatter(rows_v, [pv, cols], src, mask=mk)

        pltpu.sync_copy(rows_v, out_hbm.at[pl.ds(base, CHUNK)])

    return _soft_embed


def kernel(tokens, wte_weight, learned_embedding):
    B, S = tokens.shape
    V, D = wte_weight.shape
    N = learned_embedding.shape[0]
    k = _build(B * S, V, D, N, S)
    out = k(tokens.reshape(-1).astype(jnp.int32),
            wte_weight.astype(jnp.float32),
            learned_embedding.astype(jnp.float32))
    return out.reshape(B, S, D)


# speedup vs baseline: 1.8367x; 1.8367x over previous
import functools

import jax
import jax.numpy as jnp
from jax import lax
from jax.experimental import pallas as pl
from jax.experimental.pallas import tpu as pltpu
from jax.experimental.pallas import tpu_sc as plsc

_NC = 2
_NS = 16
_NW = _NC * _NS
_L = 16
_BIG = 1 << 20


@functools.lru_cache(maxsize=None)
def _build(T, V, D, N, S):
    CHUNK = T // _NW
    HALF = 128
    NSUB = CHUNK // HALF
    NG = CHUNK // _L
    NCOL = D // _L
    assert T % _NW == 0 and CHUNK % HALF == 0 and D % _L == 0
    assert S % CHUNK == 0
    assert N <= _L

    mesh = plsc.VectorSubcoreMesh(core_axis_name="c", subcore_axis_name="s")

    @functools.partial(
        pl.kernel,
        mesh=mesh,
        compiler_params=pltpu.CompilerParams(needs_layout_passes=False),
        out_type=jax.ShapeDtypeStruct((T, D), jnp.float32),
        scratch_types=[
            pltpu.VMEM((CHUNK,), jnp.int32),
            pltpu.VMEM((_L,), jnp.int32),
            pltpu.VMEM((N, D), jnp.float32),
            pltpu.VMEM((CHUNK, D), jnp.float32),
            pltpu.SemaphoreType.DMA,
        ],
    )
    def _soft_embed(tok_hbm, wte_hbm, learned_hbm, out_hbm,
                    tok_v, look_v, learned_v, rows_v, sem):
        wid = lax.axis_index("s") * _NC + lax.axis_index("c")
        base = wid * CHUNK

        pltpu.sync_copy(tok_hbm.at[pl.ds(base, CHUNK)], tok_v)

        @pl.when(wid < _NW - 1)
        def _():
            pltpu.sync_copy(tok_hbm.at[pl.ds(base + CHUNK, _L)], look_v)

        @pl.when(wid == _NW - 1)
        def _():
            look_v[...] = jnp.ones((_L,), jnp.int32)

        cps = [
            pltpu.async_copy(
                wte_hbm.at[tok_v.at[pl.ds(h * HALF, HALF)]],
                rows_v.at[pl.ds(h * HALF, HALF)],
                sem,
            )
            for h in range(NSUB)
        ]

        pltpu.sync_copy(learned_hbm, learned_v)

        lane = lax.iota(jnp.int32, _L)
        loc_v = jnp.full((_L,), _BIG, jnp.int32)
        for q in range(NG):
            t = tok_v[pl.ds(q * _L, _L)]
            m = t == 0
            c = plsc.all_reduce_population_count(m)
            f = plsc.all_reduce_ffs(m)
            loc_v = jnp.minimum(loc_v, jnp.where(c > 0, f + q * _L, _BIG))
        t = look_v[...]
        m = jnp.logical_and(t == 0, lane < N)
        c = plsc.all_reduce_population_count(m)
        f = plsc.all_reduce_ffs(m)
        loc_v = jnp.minimum(loc_v, jnp.where(c > 0, f + CHUNK, _BIG))

        for cp in cps:
            cp.wait()

        for o in range(N):
            pv = loc_v - (N - o)
            mk = jnp.logical_and(pv >= 0, pv < CHUNK)
            for cg in range(NCOL):
                cols = lane + cg * _L
                src = learned_v[o, pl.ds(cg * _L, _L)]
                plsc.store_scatter(rows_v, [pv, cols], src, mask=mk)

        pltpu.sync_copy(rows_v, out_hbm.at[pl.ds(base, CHUNK)])

    return _soft_embed


def kernel(tokens, wte_weight, learned_embedding):
    B, S = tokens.shape
    V, D = wte_weight.shape
    N = learned_embedding.shape[0]
    k = _build(B * S, V, D, N, S)
    out = k(tokens.reshape(-1).astype(jnp.int32),
            wte_weight.astype(jnp.float32),
            learned_embedding.astype(jnp.float32))
    return out.reshape(B, S, D)
